# baseline (device time: 12927 ns/iter reference)
import os

import jax
import jax.numpy as jnp
from jax import lax
from jax.experimental import pallas as pl
from jax.experimental.pallas import tpu as pltpu

N_DEV = 8
_ABLATE = os.environ.get("ABLATE", "")


def _pack(v):
    r = v.shape[0]
    return jnp.concatenate([v[: r // 2], v[r // 2 :]], axis=1)


def _unpack(v2):
    c = v2.shape[1] // 2
    return jnp.concatenate([v2[:, :c], v2[:, c:]], axis=0)


def _bitonic_stages_packed(v2, k_first, k_last, row_offset=0, dir_asc=None):
    r2, c2 = v2.shape
    c = c2 // 2
    lane = lax.broadcasted_iota(jnp.int32, v2.shape, 1)
    row = (
        lax.broadcasted_iota(jnp.int32, v2.shape, 0)
        + jnp.where(lane >= c, r2, 0)
        + row_offset
    )
    k = k_first
    while k <= k_last:
        asc = (row & k) == 0
        if dir_asc is not None:
            asc = asc == dir_asc
        j = k // 2
        while j >= 1:
            lower = (row & j) == 0
            if j == r2:
                partner = jnp.roll(v2, c, axis=1)
            else:
                down = jnp.roll(v2, -j, axis=0)
                up = jnp.roll(v2, j, axis=0)
                partner = jnp.where(lower, down, up)
            take_min = lower == asc
            v2 = jnp.where(take_min, jnp.minimum(v2, partner),
                           jnp.maximum(v2, partner))
            j //= 2
        k *= 2
    return v2


def kernel(x):
    m_per, n = x.shape
    n_total = N_DEV * m_per

    def body(x_ref, out_ref, gbuf_ref, sendbuf_ref, send_sems, recv_sems):
        my = lax.axis_index("i")

        if _ABLATE != "nocomm":
            barrier_sem = pltpu.get_barrier_semaphore()
            for off in range(1, N_DEV):
                pl.semaphore_signal(
                    barrier_sem, inc=1,
                    device_id=((my + off) % N_DEV,),
                    device_id_type=pl.DeviceIdType.MESH,
                )

        dir_asc = (my % 2) == 0
        if _ABLATE in ("nosort", "nolocal"):
            v_loc = x_ref[:, :].astype(jnp.bfloat16)
        else:
            v_loc = _unpack(_bitonic_stages_packed(
                _pack(x_ref[:, :].astype(jnp.bfloat16)), 2, m_per,
                dir_asc=dir_asc,
            ))
        gbuf_ref[pl.ds(my * m_per, m_per), :] = v_loc
        sendbuf_ref[:, :] = v_loc

        def merge(k, idx):
            base = idx * k
            blk = gbuf_ref[pl.ds(base, k), :]
            v2 = _bitonic_stages_packed(_pack(blk), k, k, row_offset=base)
            gbuf_ref[pl.ds(base, k), :] = _unpack(v2)

        def wait_from(src_dev):
            if _ABLATE == "nordma":
                return
            slot = gbuf_ref.at[pl.ds(src_dev * m_per, m_per), :]
            pltpu.make_async_remote_copy(
                src_ref=slot, dst_ref=slot,
                send_sem=send_sems.at[0],
                recv_sem=recv_sems.at[src_dev],
                device_id=(src_dev,),
                device_id_type=pl.DeviceIdType.MESH,
            ).wait_recv()

        sends = []
        if _ABLATE != "nocomm":
            pl.semaphore_wait(barrier_sem, N_DEV - 1)

        if _ABLATE not in ("nocomm", "nordma"):
            my_slot = gbuf_ref.at[pl.ds(my * m_per, m_per), :]
            for i, xor in enumerate(range(1, N_DEV)):
                rdma = pltpu.make_async_remote_copy(
                    src_ref=sendbuf_ref,
                    dst_ref=my_slot,
                    send_sem=send_sems.at[i],
                    recv_sem=recv_sems.at[my],
                    device_id=(my ^ xor,),
                    device_id_type=pl.DeviceIdType.MESH,
                )
                rdma.start()
                sends.append(rdma)

        if _ABLATE in ("nosort", "nomerge"):
            if _ABLATE == "nosort":
                for off in range(1, N_DEV):
                    wait_from((my + off) % N_DEV)
        elif _ABLATE == "nocomm":
            for k, n_blk in ((256, 4), (512, 2), (1024, 1)):
                for idx in range(n_blk):
                    merge(k, idx)
        else:
            pair = my // 2
            quad = my // 4
            wait_from(my ^ 1)
            merge(256, pair)
            wait_from(my ^ 2)
            wait_from(my ^ 3)
            merge(256, pair ^ 1)
            merge(512, quad)
            wait_from(my ^ 4)
            wait_from(my ^ 5)
            merge(256, pair ^ 2)
            wait_from(my ^ 6)
            wait_from(my ^ 7)
            merge(256, pair ^ 3)
            merge(512, quad ^ 1)
            merge(1024, 0)

        out_ref[:, :] = gbuf_ref[pl.ds(my * m_per, m_per), :].astype(jnp.float32)

        for rdma in sends:
            rdma.wait_send()

    return pl.pallas_call(
        body,
        out_shape=jax.ShapeDtypeStruct((m_per, n), jnp.float32),
        in_specs=[pl.BlockSpec(memory_space=pltpu.VMEM)],
        out_specs=pl.BlockSpec(memory_space=pltpu.VMEM),
        scratch_shapes=[
            pltpu.VMEM((n_total, n), jnp.bfloat16),
            pltpu.VMEM((m_per, n), jnp.bfloat16),
            pltpu.SemaphoreType.DMA((N_DEV - 1,)),
            pltpu.SemaphoreType.DMA((N_DEV,)),
        ],
        compiler_params=(
            None if _ABLATE == "nocomm"
            else pltpu.CompilerParams(collective_id=0)
        ),
    )(x)


# device time: 6186 ns/iter; 2.0897x vs baseline; 2.0897x over previous
import os

import jax
import jax.numpy as jnp
from jax import lax
from jax.experimental import pallas as pl
from jax.experimental.pallas import tpu as pltpu

N_DEV = 8
_ABLATE = os.environ.get("ABLATE", "")


def _pack(v):
    r = v.shape[0]
    return jnp.concatenate([v[: r // 2], v[r // 2 :]], axis=1)


def _unpack(v2):
    c = v2.shape[1] // 2
    return jnp.concatenate([v2[:, :c], v2[:, c:]], axis=0)


def _bitonic_stages_packed(v2, k_first, k_last, row_offset=0, dir_asc=None):
    r2, c2 = v2.shape
    c = c2 // 2
    lane = lax.broadcasted_iota(jnp.int32, v2.shape, 1)
    row = (
        lax.broadcasted_iota(jnp.int32, v2.shape, 0)
        + jnp.where(lane >= c, r2, 0)
        + row_offset
    )
    k = k_first
    while k <= k_last:
        asc = (row & k) == 0
        if dir_asc is not None:
            asc = asc == dir_asc
        j = k // 2
        while j >= 1:
            lower = (row & j) == 0
            if j == r2:
                partner = pltpu.roll(v2, c, axis=1)
            else:
                down = pltpu.roll(v2, r2 - j, axis=0)
                up = pltpu.roll(v2, j, axis=0)
                partner = jnp.where(lower, down, up)
            take_min = lower == asc
            v2 = jnp.where(take_min, jnp.minimum(v2, partner),
                           jnp.maximum(v2, partner))
            j //= 2
        k *= 2
    return v2


def kernel(x):
    m_per, n = x.shape
    n_total = N_DEV * m_per

    def body(x_ref, out_ref, gbuf_ref, sendbuf_ref, send_sems, recv_sems):
        my = lax.axis_index("i")

        if _ABLATE != "nocomm":
            barrier_sem = pltpu.get_barrier_semaphore()
            for off in range(1, N_DEV):
                pl.semaphore_signal(
                    barrier_sem, inc=1,
                    device_id=((my + off) % N_DEV,),
                    device_id_type=pl.DeviceIdType.MESH,
                )

        dir_asc = (my % 2) == 0
        if _ABLATE in ("nosort", "nolocal"):
            v_loc = x_ref[:, :].astype(jnp.bfloat16)
        else:
            v_loc = _unpack(_bitonic_stages_packed(
                _pack(x_ref[:, :].astype(jnp.bfloat16)), 2, m_per,
                dir_asc=dir_asc,
            ))
        gbuf_ref[pl.ds(my * m_per, m_per), :] = v_loc
        sendbuf_ref[:, :] = v_loc

        def merge(k, idx):
            base = idx * k
            blk = gbuf_ref[pl.ds(base, k), :]
            v2 = _bitonic_stages_packed(_pack(blk), k, k, row_offset=base)
            gbuf_ref[pl.ds(base, k), :] = _unpack(v2)

        def wait_from(src_dev):
            if _ABLATE == "nordma":
                return
            slot = gbuf_ref.at[pl.ds(src_dev * m_per, m_per), :]
            pltpu.make_async_remote_copy(
                src_ref=slot, dst_ref=slot,
                send_sem=send_sems.at[0],
                recv_sem=recv_sems.at[src_dev],
                device_id=(src_dev,),
                device_id_type=pl.DeviceIdType.MESH,
            ).wait_recv()

        sends = []
        if _ABLATE != "nocomm":
            pl.semaphore_wait(barrier_sem, N_DEV - 1)

        if _ABLATE not in ("nocomm", "nordma"):
            my_slot = gbuf_ref.at[pl.ds(my * m_per, m_per), :]
            for i, xor in enumerate(range(1, N_DEV)):
                rdma = pltpu.make_async_remote_copy(
                    src_ref=sendbuf_ref,
                    dst_ref=my_slot,
                    send_sem=send_sems.at[i],
                    recv_sem=recv_sems.at[my],
                    device_id=(my ^ xor,),
                    device_id_type=pl.DeviceIdType.MESH,
                )
                rdma.start()
                sends.append(rdma)

        if _ABLATE in ("nosort", "nomerge"):
            if _ABLATE == "nosort":
                for off in range(1, N_DEV):
                    wait_from((my + off) % N_DEV)
        elif _ABLATE == "nocomm":
            for k, n_blk in ((256, 4), (512, 2), (1024, 1)):
                for idx in range(n_blk):
                    merge(k, idx)
        else:
            pair = my // 2
            quad = my // 4
            wait_from(my ^ 1)
            merge(256, pair)
            wait_from(my ^ 2)
            wait_from(my ^ 3)
            merge(256, pair ^ 1)
            merge(512, quad)
            wait_from(my ^ 4)
            wait_from(my ^ 5)
            merge(256, pair ^ 2)
            wait_from(my ^ 6)
            wait_from(my ^ 7)
            merge(256, pair ^ 3)
            merge(512, quad ^ 1)
            merge(1024, 0)

        out_ref[:, :] = gbuf_ref[pl.ds(my * m_per, m_per), :].astype(jnp.float32)

        for rdma in sends:
            rdma.wait_send()

    return pl.pallas_call(
        body,
        out_shape=jax.ShapeDtypeStruct((m_per, n), jnp.float32),
        in_specs=[pl.BlockSpec(memory_space=pltpu.VMEM)],
        out_specs=pl.BlockSpec(memory_space=pltpu.VMEM),
        scratch_shapes=[
            pltpu.VMEM((n_total, n), jnp.bfloat16),
            pltpu.VMEM((m_per, n), jnp.bfloat16),
            pltpu.SemaphoreType.DMA((N_DEV - 1,)),
            pltpu.SemaphoreType.DMA((N_DEV,)),
        ],
        compiler_params=(
            None if _ABLATE == "nocomm"
            else pltpu.CompilerParams(collective_id=0)
        ),
    )(x)
